# baseline (speedup 1.0000x reference)
import functools

import jax
import jax.numpy as jnp
from jax import lax
from jax.experimental import pallas as pl
from jax.experimental.pallas import tpu as pltpu
from jax.experimental.pallas import tpu_sc as plsc

_BLK = 128
_NC = 2
_NS = 16


def _sc_segment_sum(x, edge_blocks, zeros):
  N, D = x.shape
  NP = zeros.shape[0]
  NW = _NC * _NS
  rows_per_tile = NP // _NS
  nmine = edge_blocks.shape[1]
  assert edge_blocks.shape[0] == NW

  mesh = plsc.VectorSubcoreMesh(core_axis_name="c", subcore_axis_name="s")

  @functools.partial(
      pl.kernel,
      out_type=jax.ShapeDtypeStruct((_NC, NP, D), jnp.float32),
      mesh=mesh,
      scratch_types=[
          pltpu.VMEM_SHARED((NP, D), jnp.float32),
          pltpu.VMEM((2, 2, _BLK), jnp.int32),
          pltpu.VMEM((2, _BLK, D), jnp.float32),
          pltpu.SemaphoreType.DMA,
      ],
  )
  def k(x_hbm, eb_hbm, z_hbm, out_hbm, acc, eb_v, rows_v, gsem):
    c = lax.axis_index("c")
    s = lax.axis_index("s")
    wid = s * _NC + c
    r0 = s * rows_per_tile

    pltpu.sync_copy(z_hbm.at[pl.ds(r0, rows_per_tile)],
                    acc.at[pl.ds(r0, rows_per_tile)])
    plsc.subcore_barrier()

    pltpu.sync_copy(eb_hbm.at[wid, 0], eb_v.at[0])
    pltpu.async_copy(x_hbm.at[eb_v.at[0, 0]], rows_v.at[0], gsem)

    @pl.loop(0, nmine)
    def _(b):
      jc = lax.rem(b, 2)
      jn = 1 - jc

      @pl.when(b < nmine - 1)
      def _():
        pltpu.sync_copy(eb_hbm.at[wid, b + 1], eb_v.at[jn])
        pltpu.async_copy(x_hbm.at[eb_v.at[jn, 0]], rows_v.at[jn], gsem)

      pltpu.make_async_copy(x_hbm.at[eb_v.at[jc, 0]], rows_v.at[jc],
                            gsem).wait()
      pltpu.sync_copy(rows_v.at[jc], acc.at[eb_v.at[jc, 1]], add=True)

    plsc.subcore_barrier()
    pltpu.sync_copy(acc.at[pl.ds(r0, rows_per_tile)],
                    out_hbm.at[c, pl.ds(r0, rows_per_tile)])

  return k(x, edge_blocks, zeros)


def _tc_self(x, w_s):
  N, D = x.shape
  OUT = w_s.shape[1]
  BN = 1000

  def body(x_ref, ws_ref, o_ref):
    o_ref[...] = jnp.dot(x_ref[...], ws_ref[...],
                         preferred_element_type=jnp.float32)

  return pl.pallas_call(
      body,
      grid=(N // BN,),
      in_specs=[
          pl.BlockSpec((BN, D), lambda i: (i, 0)),
          pl.BlockSpec((D, OUT), lambda i: (0, 0)),
      ],
      out_specs=pl.BlockSpec((BN, OUT), lambda i: (i, 0)),
      out_shape=jax.ShapeDtypeStruct((N, OUT), jnp.float32),
  )(x, w_s)


def _tc_finish(self_term, partials, w_n):
  N, OUT = self_term.shape
  D = w_n.shape[0]
  BN = 1000

  def body(s_ref, p_ref, wn_ref, o_ref):
    agg = p_ref[0] + p_ref[1]
    o_ref[...] = jnp.maximum(
        s_ref[...]
        + jnp.dot(agg, wn_ref[...], preferred_element_type=jnp.float32),
        0.0)

  return pl.pallas_call(
      body,
      grid=(N // BN,),
      in_specs=[
          pl.BlockSpec((BN, OUT), lambda i: (i, 0)),
          pl.BlockSpec((_NC, BN, D), lambda i: (0, i, 0)),
          pl.BlockSpec((D, OUT), lambda i: (0, 0)),
      ],
      out_specs=pl.BlockSpec((BN, OUT), lambda i: (i, 0)),
      out_shape=jax.ShapeDtypeStruct((N, OUT), jnp.float32),
  )(self_term, partials, w_n)


def kernel(x, edge_index, w_s, w_n):
  N, D = x.shape
  E = edge_index.shape[1]
  align = 8 * _NS
  NP = ((N + align - 1) // align) * align
  if NP == N:
    NP += align

  NW = _NC * _NS
  quantum = NW * _BLK
  EP = ((E + quantum - 1) // quantum) * quantum
  pad = EP - E
  if pad:
    pad_dst = N + jnp.arange(pad, dtype=jnp.int32) % (NP - N)
    pad_block = jnp.stack(
        [jnp.zeros((pad,), jnp.int32), pad_dst], axis=0)
    edge_index = jnp.concatenate([edge_index, pad_block], axis=1)
  nmine = EP // quantum
  edge_blocks = (edge_index.reshape(2, NW, nmine, _BLK)
                 .transpose(1, 2, 0, 3))
  zeros = jnp.zeros((NP, D), jnp.float32)
  self_term = _tc_self(x, w_s)
  partials = _sc_segment_sum(x, edge_blocks, zeros)
  return _tc_finish(self_term, partials, w_n)


# speedup vs baseline: 1.6868x; 1.6868x over previous
import functools

import jax
import jax.numpy as jnp
from jax import lax
from jax.experimental import pallas as pl
from jax.experimental.pallas import tpu as pltpu
from jax.experimental.pallas import tpu_sc as plsc

_BLK = 128
_NC = 2
_NS = 16


def _sc_segment_sum(x, edge_blocks, zeros):
  N, D = x.shape
  NP = zeros.shape[0]
  NW = _NC * _NS
  rows_per_tile = NP // _NS
  NB = edge_blocks.shape[0]
  base, rem = NB // NW, NB % NW

  mesh = plsc.VectorSubcoreMesh(core_axis_name="c", subcore_axis_name="s")

  @functools.partial(
      pl.kernel,
      out_type=jax.ShapeDtypeStruct((_NC, NP, D), jnp.float32),
      mesh=mesh,
      scratch_types=[
          pltpu.VMEM_SHARED((NP, D), jnp.float32),
          pltpu.VMEM((2, 2, _BLK), jnp.int32),
          pltpu.VMEM((_BLK, D), jnp.float32),
          pltpu.SemaphoreType.DMA,
          pltpu.SemaphoreType.DMA,
      ],
  )
  def k(x_hbm, eb_hbm, z_hbm, out_hbm, acc, eb_v, rows_v, esem, gsem):
    c = lax.axis_index("c")
    s = lax.axis_index("s")
    wid = s * _NC + c
    r0 = s * rows_per_tile

    pltpu.sync_copy(z_hbm.at[pl.ds(r0, rows_per_tile)],
                    acc.at[pl.ds(r0, rows_per_tile)])
    plsc.subcore_barrier()

    nmine = base + jnp.where(wid < rem, 1, 0)
    pltpu.async_copy(eb_hbm.at[wid], eb_v.at[0], esem)

    @pl.loop(0, nmine)
    def _(kk):
      jc = lax.rem(kk, 2)
      jn = 1 - jc
      pltpu.make_async_copy(eb_hbm.at[wid], eb_v.at[jc], esem).wait()

      @pl.when(kk < nmine - 1)
      def _():
        pltpu.async_copy(eb_hbm.at[wid + NW * (kk + 1)], eb_v.at[jn], esem)

      pltpu.async_copy(x_hbm.at[eb_v.at[jc, 0]], rows_v, gsem).wait()
      pltpu.sync_copy(rows_v, acc.at[eb_v.at[jc, 1]], add=True)

    plsc.subcore_barrier()
    pltpu.sync_copy(acc.at[pl.ds(r0, rows_per_tile)],
                    out_hbm.at[c, pl.ds(r0, rows_per_tile)])

  return k(x, edge_blocks, zeros)


def _tc_self(x, w_s):
  N, D = x.shape
  OUT = w_s.shape[1]
  BN = 1000

  def body(x_ref, ws_ref, o_ref):
    o_ref[...] = jnp.dot(x_ref[...], ws_ref[...],
                         preferred_element_type=jnp.float32)

  return pl.pallas_call(
      body,
      grid=(N // BN,),
      in_specs=[
          pl.BlockSpec((BN, D), lambda i: (i, 0)),
          pl.BlockSpec((D, OUT), lambda i: (0, 0)),
      ],
      out_specs=pl.BlockSpec((BN, OUT), lambda i: (i, 0)),
      out_shape=jax.ShapeDtypeStruct((N, OUT), jnp.float32),
  )(x, w_s)


def _tc_finish(self_term, partials, w_n):
  N, OUT = self_term.shape
  D = w_n.shape[0]
  BN = 1000

  def body(s_ref, p_ref, wn_ref, o_ref):
    agg = p_ref[0] + p_ref[1]
    o_ref[...] = jnp.maximum(
        s_ref[...]
        + jnp.dot(agg, wn_ref[...], preferred_element_type=jnp.float32),
        0.0)

  return pl.pallas_call(
      body,
      grid=(N // BN,),
      in_specs=[
          pl.BlockSpec((BN, OUT), lambda i: (i, 0)),
          pl.BlockSpec((_NC, BN, D), lambda i: (0, i, 0)),
          pl.BlockSpec((D, OUT), lambda i: (0, 0)),
      ],
      out_specs=pl.BlockSpec((BN, OUT), lambda i: (i, 0)),
      out_shape=jax.ShapeDtypeStruct((N, OUT), jnp.float32),
  )(self_term, partials, w_n)


def kernel(x, edge_index, w_s, w_n):
  N, D = x.shape
  E = edge_index.shape[1]
  align = 8 * _NS
  NP = ((N + align - 1) // align) * align
  if NP == N:
    NP += align

  NW = _NC * _NS
  assert E % _BLK == 0
  NB = E // _BLK
  edge_blocks = edge_index.reshape(2, NB, _BLK).transpose(1, 0, 2)
  zeros = jnp.zeros((NP, D), jnp.float32)
  self_term = _tc_self(x, w_s)
  partials = _sc_segment_sum(x, edge_blocks, zeros)
  return _tc_finish(self_term, partials, w_n)


# speedup vs baseline: 1.6937x; 1.0041x over previous
import functools

import jax
import jax.numpy as jnp
from jax import lax
from jax.experimental import pallas as pl
from jax.experimental.pallas import tpu as pltpu
from jax.experimental.pallas import tpu_sc as plsc

_BLK = 128
_NC = 2
_NS = 16


def _sc_segment_sum(x, edge_blocks, zeros):
  N, D = x.shape
  NP = zeros.shape[0]
  NW = _NC * _NS
  rows_per_tile = NP // _NS
  NB = edge_blocks.shape[0]
  base, rem = NB // NW, NB % NW

  mesh = plsc.VectorSubcoreMesh(core_axis_name="c", subcore_axis_name="s")

  @functools.partial(
      pl.kernel,
      out_type=jax.ShapeDtypeStruct((_NC, NP, D), jnp.float32),
      mesh=mesh,
      scratch_types=[
          pltpu.VMEM_SHARED((NP, D), jnp.float32),
          pltpu.VMEM((2, 2, _BLK), jnp.int32),
          pltpu.VMEM((_BLK, D), jnp.float32),
          pltpu.SemaphoreType.DMA,
          pltpu.SemaphoreType.DMA,
      ],
  )
  def k(x_hbm, eb_hbm, z_hbm, out_hbm, acc, eb_v, rows_v, esem, gsem):
    c = lax.axis_index("c")
    s = lax.axis_index("s")
    wid = s * _NC + c
    r0 = s * rows_per_tile

    pltpu.sync_copy(z_hbm.at[pl.ds(r0, rows_per_tile)],
                    acc.at[pl.ds(r0, rows_per_tile)])
    plsc.subcore_barrier()

    nmine = base + jnp.where(wid < rem, 1, 0)
    pltpu.async_copy(eb_hbm.at[wid], eb_v.at[0], esem)

    @pl.loop(0, nmine)
    def _(kk):
      jc = lax.rem(kk, 2)
      jn = 1 - jc
      pltpu.make_async_copy(eb_hbm.at[wid], eb_v.at[jc], esem).wait()

      @pl.when(kk < nmine - 1)
      def _():
        pltpu.async_copy(eb_hbm.at[wid + NW * (kk + 1)], eb_v.at[jn], esem)

      pltpu.async_copy(x_hbm.at[eb_v.at[jc, 0]], rows_v, gsem).wait()
      pltpu.sync_copy(rows_v, acc.at[eb_v.at[jc, 1]], add=True)

    plsc.subcore_barrier()
    pltpu.sync_copy(acc.at[pl.ds(r0, rows_per_tile)],
                    out_hbm.at[c, pl.ds(r0, rows_per_tile)])

  return k(x, edge_blocks, zeros)


def _tc_finish(x, partials, w_s, w_n):
  N, D = x.shape
  OUT = w_s.shape[1]
  BN = 1000

  def body(x_ref, p_ref, ws_ref, wn_ref, o_ref):
    agg = p_ref[0] + p_ref[1]
    o_ref[...] = jnp.maximum(
        jnp.dot(x_ref[...], ws_ref[...], preferred_element_type=jnp.float32)
        + jnp.dot(agg, wn_ref[...], preferred_element_type=jnp.float32),
        0.0)

  return pl.pallas_call(
      body,
      grid=(N // BN,),
      in_specs=[
          pl.BlockSpec((BN, D), lambda i: (i, 0)),
          pl.BlockSpec((_NC, BN, D), lambda i: (0, i, 0)),
          pl.BlockSpec((D, OUT), lambda i: (0, 0)),
          pl.BlockSpec((D, OUT), lambda i: (0, 0)),
      ],
      out_specs=pl.BlockSpec((BN, OUT), lambda i: (i, 0)),
      out_shape=jax.ShapeDtypeStruct((N, OUT), jnp.float32),
  )(x, partials, w_s, w_n)


def kernel(x, edge_index, w_s, w_n):
  N, D = x.shape
  E = edge_index.shape[1]
  align = 8 * _NS
  NP = ((N + align - 1) // align) * align
  if NP == N:
    NP += align

  NW = _NC * _NS
  assert E % _BLK == 0
  NB = E // _BLK
  edge_blocks = edge_index.reshape(2, NB, _BLK).transpose(1, 0, 2)
  zeros = jnp.zeros((NP, D), jnp.float32)
  partials = _sc_segment_sum(x, edge_blocks, zeros)
  return _tc_finish(x, partials, w_s, w_n)
